# baseline (speedup 1.0000x reference)
import jax
import jax.numpy as jnp
from jax import lax
from jax.experimental import pallas as pl
from jax.experimental.pallas import tpu as pltpu
from jax.experimental.pallas import tpu_sc as plsc

B, L, H = 16, 1024, 768
HALF = L // 2
CHUNK = 16
NCHUNK = HALF // CHUNK
NSLOT = 4
LANES = 16
GROWS = 8


def _positions_body(x_ref, out_ref):
    mask = (x_ref[...] != 0)
    tri = (lax.broadcasted_iota(jnp.int32, (L, L), 0)
           <= lax.broadcasted_iota(jnp.int32, (L, L), 1)).astype(jnp.float32)
    cs = jax.lax.dot_general(mask.astype(jnp.float32), tri,
                             (((1,), (0,)), ((), ())),
                             preferred_element_type=jnp.float32)
    out_ref[...] = cs.astype(jnp.int32) * mask.astype(jnp.int32)


def _positions(x):
    return pl.pallas_call(
        _positions_body,
        out_shape=jax.ShapeDtypeStruct((B, L), jnp.int32),
    )(x)


def _sc_body(x_hbm, posn_hbm, atom_hbm, pos_hbm, gt_hbm, out_hbm,
             aidx, pidx, oidx, gidx, gtbuf, *bufsem):
    c = lax.axis_index("c")
    s = lax.axis_index("s")
    b = s
    half = c
    iota = lax.iota(jnp.int32, LANES)

    abufs = bufsem[0:NSLOT]
    pbufs = bufsem[NSLOT:2 * NSLOT]
    semas = bufsem[2 * NSLOT:3 * NSLOT]
    semps = bufsem[3 * NSLOT:4 * NSLOT]
    semos = bufsem[4 * NSLOT:5 * NSLOT]
    semg = bufsem[5 * NSLOT]

    base = 1 + half * HALF
    sa = pltpu.async_copy(x_hbm.at[b, pl.ds(half * HALF, HALF)], aidx, semas[0])
    sp = pltpu.async_copy(posn_hbm.at[b, pl.ds(half * HALF, HALF)], pidx, semps[0])
    for k in range(NCHUNK):
        for j in range(CHUNK // LANES):
            oidx[k, pl.ds(j * LANES, LANES)] = base + k * CHUNK + j * LANES + iota
    sa.wait()
    sp.wait()

    gath = [None] * NSLOT
    scat = [None] * NSLOT

    def issue(kk):
        sl = kk % NSLOT
        if scat[sl] is not None:
            scat[sl].wait()
            scat[sl] = None
        gath[sl] = (
            pltpu.async_copy(atom_hbm.at[aidx.at[pl.ds(kk * CHUNK, CHUNK)]],
                             abufs[sl], semas[sl]),
            pltpu.async_copy(pos_hbm.at[pidx.at[pl.ds(kk * CHUNK, CHUNK)]],
                             pbufs[sl], semps[sl]))

    for kk in range(min(NSLOT - 1, NCHUNK)):
        issue(kk)
    for k in range(NCHUNK):
        slot = k % NSLOT
        if k + NSLOT - 1 < NCHUNK:
            issue(k + NSLOT - 1)
        ga, gp = gath[slot]
        ga.wait()
        gp.wait()
        ab, pb = abufs[slot], pbufs[slot]

        def add_row(t, _, ab=ab, pb=pb):
            def add_col(j, __):
                sl = pl.ds(j * LANES, LANES)
                ab[t, sl] = ab[t, sl] + pb[t, sl]
                return 0
            lax.fori_loop(0, H // LANES, add_col, 0, unroll=8)
            return 0

        lax.fori_loop(0, CHUNK, add_row, 0)
        scat[slot] = pltpu.async_copy(ab, out_hbm.at[b].at[oidx.at[k]], semos[slot])
    for sl in range(NSLOT):
        if scat[sl] is not None:
            scat[sl].wait()

    gidx[pl.ds(0, LANES)] = iota * 0
    pltpu.async_copy(gt_hbm.at[gidx], gtbuf, semg).wait()
    pltpu.async_copy(gtbuf, out_hbm.at[b].at[gidx], semg).wait()


def kernel(x, atom_table, pos_table, graph_token):
    positions = _positions(x)
    mesh = plsc.VectorSubcoreMesh(
        core_axis_name="c", subcore_axis_name="s", num_cores=2, num_subcores=16)
    f = pl.kernel(
        _sc_body,
        out_type=jax.ShapeDtypeStruct((B, L + 1, H), jnp.float32),
        mesh=mesh,
        scratch_types=[
            pltpu.VMEM((HALF,), jnp.int32),
            pltpu.VMEM((HALF,), jnp.int32),
            pltpu.VMEM((NCHUNK, CHUNK), jnp.int32),
            pltpu.VMEM((LANES,), jnp.int32),
            pltpu.VMEM((LANES, H), jnp.float32),
        ] + [pltpu.VMEM((CHUNK, H), jnp.float32) for _ in range(2 * NSLOT)]
          + [pltpu.SemaphoreType.DMA for _ in range(3 * NSLOT + 1)],
    )
    return f(x, positions, atom_table, pos_table, graph_token)
